# baseline (speedup 1.0000x reference)
import jax
import jax.numpy as jnp
from jax.experimental import pallas as pl
from jax.experimental.pallas import tpu as pltpu

B, L, T, KH, DH = 16, 2, 2048, 8, 64
ROW = 2 * KH * DH
M = B * L
TH = T // 2
NCH = M * 2
NBUF = 8
DLAG = 4


def _body(layer_ref, idx_ref, kv_ref, buf_any, out_any, vbuf, in_sems, out_sems, row_sem):
    layer = layer_ref[0]
    idx = idx_ref[0]

    def in_dma(i, s):
        return pltpu.make_async_copy(buf_any.at[pl.ds(i, 1)], vbuf.at[s], in_sems.at[s])

    def out_dma(i, s):
        return pltpu.make_async_copy(vbuf.at[s], out_any.at[pl.ds(i, 1)], out_sems.at[s])

    for i in range(NCH + DLAG):
        if i < NCH:
            s = i % NBUF
            if i >= NBUF:
                out_dma(i - NBUF, s).wait()
            in_dma(i, s).start()
        if i >= DLAG:
            j = i - DLAG
            sw = j % NBUF
            in_dma(j, sw).wait()
            out_dma(j, sw).start()
    for j in range(NCH - NBUF, NCH):
        out_dma(j, j % NBUF).wait()

    p_hi = idx // TH
    r = idx - p_hi * TH
    for b in range(B):
        p = (b * L + layer) * 2 + p_hi
        pltpu.make_async_copy(kv_ref.at[b], out_any.at[p, r], row_sem).start()
    for b in range(B):
        p = (b * L + layer) * 2 + p_hi
        pltpu.make_async_copy(kv_ref.at[b], out_any.at[p, r], row_sem).wait()


@jax.jit
def _run(layer_s, idx_s, kv, buf3):
    return pl.pallas_call(
        _body,
        in_specs=[
            pl.BlockSpec(memory_space=pltpu.SMEM),
            pl.BlockSpec(memory_space=pltpu.SMEM),
            pl.BlockSpec(memory_space=pltpu.VMEM),
            pl.BlockSpec(memory_space=pl.ANY),
        ],
        out_specs=pl.BlockSpec(memory_space=pl.ANY),
        out_shape=jax.ShapeDtypeStruct((NCH, TH, ROW), jnp.float32),
        scratch_shapes=[
            pltpu.VMEM((NBUF, 1, TH, ROW), jnp.float32),
            pltpu.SemaphoreType.DMA((NBUF,)),
            pltpu.SemaphoreType.DMA((NBUF,)),
            pltpu.SemaphoreType.DMA,
        ],
    )(layer_s, idx_s, kv, buf3)


def kernel(buf, k_step, v_step, layer, idx):
    layer = jnp.clip(jnp.asarray(layer, jnp.int32), 0, L - 1)
    idx = jnp.clip(jnp.asarray(idx, jnp.int32), 0, T - 1)
    step = jnp.clip(idx, 0, k_step.shape[1] - 1)
    ks = jax.lax.dynamic_index_in_dim(k_step, step, axis=1, keepdims=False)
    vs = jax.lax.dynamic_index_in_dim(v_step, step, axis=1, keepdims=False)
    kv = jnp.concatenate([ks.reshape(B, KH * DH), vs.reshape(B, KH * DH)], axis=1)
    out3 = _run(layer.reshape(1), idx.reshape(1), kv, buf.reshape(NCH, TH, ROW))
    return out3.reshape(B, L, T, 2, KH, DH)


# speedup vs baseline: 1.1544x; 1.1544x over previous
import functools

import jax
import jax.numpy as jnp
from jax import lax
from jax.experimental import pallas as pl
from jax.experimental.pallas import tpu as pltpu
from jax.experimental.pallas import tpu_sc as plsc

B, L, T, KH, DH = 16, 2, 2048, 8, 64
ROW = 2 * KH * DH
M = B * L

NROWS = M * T * 2
RW = 512
NW = 32
RPW = NROWS // NW
CH = 128
NCH = RPW // CH

_mesh = plsc.VectorSubcoreMesh(core_axis_name="c", subcore_axis_name="s")


@functools.partial(
    pl.kernel,
    mesh=_mesh,
    out_type=jax.ShapeDtypeStruct((NROWS, RW), jnp.float32),
    scratch_types=[
        pltpu.VMEM((2, CH, RW), jnp.float32),
        pltpu.SemaphoreType.DMA((2,)),
        pltpu.SemaphoreType.DMA((2,)),
    ],
)
def _sc_copy(buf_hbm, out_hbm, vbuf, in_sems, out_sems):
    wid = lax.axis_index("s") * 2 + lax.axis_index("c")
    base = wid * RPW

    def in_dma(k, s):
        return pltpu.make_async_copy(
            buf_hbm.at[pl.ds(base + k * CH, CH)], vbuf.at[s], in_sems.at[s]
        )

    def out_dma(k, s):
        return pltpu.make_async_copy(
            vbuf.at[s], out_hbm.at[pl.ds(base + k * CH, CH)], out_sems.at[s]
        )

    in_dma(0, 0).start()

    def body(k, _):
        s = lax.rem(k, 2)
        in_dma(k, s).wait()
        out_dma(k, s).start()

        @pl.when(k + 1 < NCH)
        def _():
            s1 = lax.rem(k + 1, 2)

            @pl.when(k + 1 >= 2)
            def _():
                out_dma(k - 1, s1).wait()

            in_dma(k + 1, s1).start()

        return 0

    lax.fori_loop(0, NCH, body, 0)
    out_dma(NCH - 2, 0 if (NCH - 2) % 2 == 0 else 1).wait()
    out_dma(NCH - 1, 0 if (NCH - 1) % 2 == 0 else 1).wait()


def _scatter_body(layer_ref, idx_ref, kv_ref, buf_any, out_any, sem):
    del buf_any
    layer = layer_ref[0]
    idx = idx_ref[0]
    for b in range(B):
        pltpu.make_async_copy(
            kv_ref.at[b], out_any.at[b * L + layer, idx], sem
        ).start()
    for b in range(B):
        pltpu.make_async_copy(
            kv_ref.at[b], out_any.at[b * L + layer, idx], sem
        ).wait()


@jax.jit
def _run(layer_s, idx_s, kv, buf2):
    copied = _sc_copy(buf2)
    copied3 = copied.reshape(M, T, ROW)
    return pl.pallas_call(
        _scatter_body,
        in_specs=[
            pl.BlockSpec(memory_space=pltpu.SMEM),
            pl.BlockSpec(memory_space=pltpu.SMEM),
            pl.BlockSpec(memory_space=pltpu.VMEM),
            pl.BlockSpec(memory_space=pl.ANY),
        ],
        out_specs=pl.BlockSpec(memory_space=pl.ANY),
        out_shape=jax.ShapeDtypeStruct((M, T, ROW), jnp.float32),
        scratch_shapes=[pltpu.SemaphoreType.DMA],
        input_output_aliases={3: 0},
    )(layer_s, idx_s, kv, copied3)


def kernel(buf, k_step, v_step, layer, idx):
    layer = jnp.clip(jnp.asarray(layer, jnp.int32), 0, L - 1)
    idx = jnp.clip(jnp.asarray(idx, jnp.int32), 0, T - 1)
    step = jnp.clip(idx, 0, k_step.shape[1] - 1)
    ks = jax.lax.dynamic_index_in_dim(k_step, step, axis=1, keepdims=False)
    vs = jax.lax.dynamic_index_in_dim(v_step, step, axis=1, keepdims=False)
    kv = jnp.concatenate([ks.reshape(B, KH * DH), vs.reshape(B, KH * DH)], axis=1)
    out3 = _run(layer.reshape(1), idx.reshape(1), kv, buf.reshape(NROWS, RW))
    return out3.reshape(B, L, T, 2, KH, DH)


# speedup vs baseline: 1.1727x; 1.0159x over previous
import functools

import jax
import jax.numpy as jnp
from jax import lax
from jax.experimental import pallas as pl
from jax.experimental.pallas import tpu as pltpu
from jax.experimental.pallas import tpu_sc as plsc

B, L, T, KH, DH = 16, 2, 2048, 8, 64
ROW = 2 * KH * DH
M = B * L

NROWS = M * T * 2
RW = 512
NW = 32
RPW = NROWS // NW
CH = 128
NCH = RPW // CH

_mesh = plsc.VectorSubcoreMesh(core_axis_name="c", subcore_axis_name="s")


@functools.partial(
    pl.kernel,
    mesh=_mesh,
    out_type=jax.ShapeDtypeStruct((NROWS, RW), jnp.float32),
    scratch_types=[
        pltpu.VMEM_SHARED((16, 2, CH, RW), jnp.float32),
        pltpu.SemaphoreType.DMA((2,)),
        pltpu.SemaphoreType.DMA((2,)),
    ],
)
def _sc_copy(buf_hbm, out_hbm, shbuf, in_sems, out_sems):
    sid = lax.axis_index("s")
    wid = sid * 2 + lax.axis_index("c")
    base = wid * RPW

    def in_dma(k, s):
        return pltpu.make_async_copy(
            buf_hbm.at[pl.ds(base + k * CH, CH)], shbuf.at[sid, s], in_sems.at[s]
        )

    def out_dma(k, s):
        return pltpu.make_async_copy(
            shbuf.at[sid, s], out_hbm.at[pl.ds(base + k * CH, CH)], out_sems.at[s]
        )

    in_dma(0, 0).start()

    def body(k, _):
        s = lax.rem(k, 2)
        in_dma(k, s).wait()
        out_dma(k, s).start()

        @pl.when(k + 1 < NCH)
        def _():
            s1 = lax.rem(k + 1, 2)

            @pl.when(k + 1 >= 2)
            def _():
                out_dma(k - 1, s1).wait()

            in_dma(k + 1, s1).start()

        return 0

    lax.fori_loop(0, NCH, body, 0)
    out_dma(NCH - 2, (NCH - 2) % 2).wait()
    out_dma(NCH - 1, (NCH - 1) % 2).wait()


def _scatter_body(layer_ref, idx_ref, kv_ref, buf_any, out_any, sem):
    del buf_any
    layer = layer_ref[0]
    idx = idx_ref[0]
    for b in range(B):
        pltpu.make_async_copy(
            kv_ref.at[b], out_any.at[b * L + layer, idx], sem
        ).start()
    for b in range(B):
        pltpu.make_async_copy(
            kv_ref.at[b], out_any.at[b * L + layer, idx], sem
        ).wait()


@jax.jit
def _run(layer_s, idx_s, kv, buf2):
    copied = _sc_copy(buf2)
    copied3 = copied.reshape(M, T, ROW)
    return pl.pallas_call(
        _scatter_body,
        in_specs=[
            pl.BlockSpec(memory_space=pltpu.SMEM),
            pl.BlockSpec(memory_space=pltpu.SMEM),
            pl.BlockSpec(memory_space=pltpu.VMEM),
            pl.BlockSpec(memory_space=pl.ANY),
        ],
        out_specs=pl.BlockSpec(memory_space=pl.ANY),
        out_shape=jax.ShapeDtypeStruct((M, T, ROW), jnp.float32),
        scratch_shapes=[pltpu.SemaphoreType.DMA],
        input_output_aliases={3: 0},
    )(layer_s, idx_s, kv, copied3)


def kernel(buf, k_step, v_step, layer, idx):
    layer = jnp.clip(jnp.asarray(layer, jnp.int32), 0, L - 1)
    idx = jnp.clip(jnp.asarray(idx, jnp.int32), 0, T - 1)
    step = jnp.clip(idx, 0, k_step.shape[1] - 1)
    ks = jax.lax.dynamic_index_in_dim(k_step, step, axis=1, keepdims=False)
    vs = jax.lax.dynamic_index_in_dim(v_step, step, axis=1, keepdims=False)
    kv = jnp.concatenate([ks.reshape(B, KH * DH), vs.reshape(B, KH * DH)], axis=1)
    out3 = _run(layer.reshape(1), idx.reshape(1), kv, buf.reshape(NROWS, RW))
    return out3.reshape(B, L, T, 2, KH, DH)


# speedup vs baseline: 1.8862x; 1.6084x over previous
import jax
import jax.numpy as jnp
from jax.experimental import pallas as pl
from jax.experimental.pallas import tpu as pltpu

B, L, T, KH, DH = 16, 2, 2048, 8, 64
HD = KH * DH


def _body(layer_ref, idx_ref, k_ref, v_ref, buf_any, out_any, ksem, vsem):
    del buf_any
    layer = layer_ref[0]
    idx = idx_ref[0]
    ck = pltpu.make_async_copy(k_ref, out_any.at[:, layer, idx, 0], ksem)
    cv = pltpu.make_async_copy(v_ref, out_any.at[:, layer, idx, 1], vsem)
    ck.start()
    cv.start()
    ck.wait()
    cv.wait()


@jax.jit
def _run(layer_s, idx_s, k2, v2, buf5):
    return pl.pallas_call(
        _body,
        in_specs=[
            pl.BlockSpec(memory_space=pltpu.SMEM),
            pl.BlockSpec(memory_space=pltpu.SMEM),
            pl.BlockSpec(memory_space=pltpu.VMEM),
            pl.BlockSpec(memory_space=pltpu.VMEM),
            pl.BlockSpec(memory_space=pl.ANY),
        ],
        out_specs=pl.BlockSpec(memory_space=pl.ANY),
        out_shape=jax.ShapeDtypeStruct((B, L, T, 2, HD), jnp.float32),
        scratch_shapes=[pltpu.SemaphoreType.DMA, pltpu.SemaphoreType.DMA],
        input_output_aliases={4: 0},
    )(layer_s, idx_s, k2, v2, buf5)


def kernel(buf, k_step, v_step, layer, idx):
    layer = jnp.clip(jnp.asarray(layer, jnp.int32), 0, L - 1)
    idx = jnp.clip(jnp.asarray(idx, jnp.int32), 0, T - 1)
    k2 = k_step.reshape(B, HD)
    v2 = v_step.reshape(B, HD)
    out5 = _run(layer.reshape(1), idx.reshape(1), k2, v2, buf.reshape(B, L, T, 2, HD))
    return out5.reshape(B, L, T, 2, KH, DH)


# speedup vs baseline: 3.2554x; 1.7259x over previous
import jax
import jax.numpy as jnp
from jax.experimental import pallas as pl
from jax.experimental.pallas import tpu as pltpu

B, L, T, KH, DH = 16, 2, 2048, 8, 64
ROW = 2 * KH * DH


def _body(layer_ref, idx_ref, kv_ref, buf_any, out_any, sem):
    del buf_any
    layer = layer_ref[0]
    idx = idx_ref[0]
    for b in range(B):
        pltpu.make_async_copy(
            kv_ref.at[b], out_any.at[b * L + layer, idx], sem
        ).start()
    for b in range(B):
        pltpu.make_async_copy(
            kv_ref.at[b], out_any.at[b * L + layer, idx], sem
        ).wait()


@jax.jit
def _run(layer_s, idx_s, kv, buf3):
    return pl.pallas_call(
        _body,
        in_specs=[
            pl.BlockSpec(memory_space=pltpu.SMEM),
            pl.BlockSpec(memory_space=pltpu.SMEM),
            pl.BlockSpec(memory_space=pltpu.VMEM),
            pl.BlockSpec(memory_space=pl.ANY),
        ],
        out_specs=pl.BlockSpec(memory_space=pl.ANY),
        out_shape=jax.ShapeDtypeStruct((B * L, T, ROW), jnp.float32),
        scratch_shapes=[pltpu.SemaphoreType.DMA],
        input_output_aliases={3: 0},
    )(layer_s, idx_s, kv, buf3)


def kernel(buf, k_step, v_step, layer, idx):
    layer = jnp.clip(jnp.asarray(layer, jnp.int32), 0, L - 1)
    idx = jnp.clip(jnp.asarray(idx, jnp.int32), 0, T - 1)
    kv = jnp.concatenate(
        [k_step.reshape(B, KH * DH), v_step.reshape(B, KH * DH)], axis=1
    )
    out3 = _run(layer.reshape(1), idx.reshape(1), kv, buf.reshape(B * L, T, ROW))
    return out3.reshape(B, L, T, 2, KH, DH)
